# baseline (speedup 1.0000x reference)
import functools

import jax
import jax.numpy as jnp
from jax.experimental import pallas as pl
from jax.experimental.pallas import tpu as pltpu

_T = 2048
_D = 1024
_E = 8
_TB = 512


def _moe_block_kernel(x_ref, wg_ref, we_ref, o_ref):
    x = x_ref[0]
    wg = wg_ref[0]
    logits = jax.lax.dot_general(
        x, wg, (((1,), (1,)), ((), ())), preferred_element_type=jnp.float32
    )
    e = logits.shape[1]
    iota = jax.lax.broadcasted_iota(jnp.int32, logits.shape, 1)
    m1 = jnp.max(logits, axis=1, keepdims=True)
    i1 = jnp.min(jnp.where(logits == m1, iota, e), axis=1, keepdims=True)
    mask1 = iota == i1
    rest = jnp.where(mask1, -jnp.inf, logits)
    m2 = jnp.max(rest, axis=1, keepdims=True)
    i2 = jnp.min(jnp.where(rest == m2, iota, e), axis=1, keepdims=True)
    mask = mask1 | (iota == i2)
    tm = jnp.where(mask, logits, 0.0)
    gates = tm / (jnp.sum(tm, axis=1, keepdims=True) + 1e-9)

    xb = x.astype(jnp.bfloat16)
    acc = jnp.zeros((x.shape[0], x.shape[1]), jnp.float32)
    for ei in range(e):
        y = jax.lax.dot_general(
            xb, we_ref[ei], (((1,), (1,)), ((), ())),
            preferred_element_type=jnp.float32,
        )
        acc = acc + gates[:, ei : ei + 1] * y
    o_ref[0] = acc


@functools.partial(jax.jit, static_argnums=())
def kernel(x0, x1, Wg0, Wg1, We):
    xs = jnp.stack([x0, x1])
    wgs = jnp.stack([Wg0, Wg1])
    out = pl.pallas_call(
        _moe_block_kernel,
        grid=(2, _T // _TB),
        in_specs=[
            pl.BlockSpec((1, _TB, _D), lambda t, b: (t, b, 0)),
            pl.BlockSpec((1, _E, _D), lambda t, b: (t, 0, 0)),
            pl.BlockSpec((_E, _D, _D), lambda t, b: (0, 0, 0)),
        ],
        out_specs=pl.BlockSpec((1, _TB, _D), lambda t, b: (t, b, 0)),
        out_shape=jax.ShapeDtypeStruct((2, _T, _D), jnp.float32),
        compiler_params=pltpu.CompilerParams(
            dimension_semantics=("parallel", "parallel"),
        ),
    )(xs, wgs, We.astype(jnp.bfloat16))
    return (out[0], out[1])


# speedup vs baseline: 1.0968x; 1.0968x over previous
import functools

import jax
import jax.numpy as jnp
from jax.experimental import pallas as pl
from jax.experimental.pallas import tpu as pltpu

_T = 2048
_D = 1024
_E = 8
_TB = 1024


def _moe_block_kernel(x_ref, wg_ref, we_ref, o_ref):
    x = x_ref[0]
    wg = wg_ref[0]
    logits = jax.lax.dot_general(
        x, wg, (((1,), (1,)), ((), ())), preferred_element_type=jnp.float32
    )
    e = logits.shape[1]
    iota = jax.lax.broadcasted_iota(jnp.int32, logits.shape, 1)
    m1 = jnp.max(logits, axis=1, keepdims=True)
    i1 = jnp.min(jnp.where(logits == m1, iota, e), axis=1, keepdims=True)
    mask1 = iota == i1
    rest = jnp.where(mask1, -jnp.inf, logits)
    m2 = jnp.max(rest, axis=1, keepdims=True)
    i2 = jnp.min(jnp.where(rest == m2, iota, e), axis=1, keepdims=True)
    mask = mask1 | (iota == i2)
    tm = jnp.where(mask, logits, 0.0)
    gates = tm / (jnp.sum(tm, axis=1, keepdims=True) + 1e-9)

    xb = x
    acc = jnp.zeros((x.shape[0], x.shape[1]), jnp.float32)
    for ei in range(e):
        y = jax.lax.dot_general(
            xb, we_ref[ei], (((1,), (1,)), ((), ())),
            preferred_element_type=jnp.float32,
        )
        acc = acc + gates[:, ei : ei + 1] * y
    o_ref[0] = acc


@functools.partial(jax.jit, static_argnums=())
def kernel(x0, x1, Wg0, Wg1, We):
    xs = jnp.stack([x0, x1])
    wgs = jnp.stack([Wg0, Wg1])
    out = pl.pallas_call(
        _moe_block_kernel,
        grid=(2, _T // _TB),
        in_specs=[
            pl.BlockSpec((1, _TB, _D), lambda t, b: (t, b, 0)),
            pl.BlockSpec((1, _E, _D), lambda t, b: (t, 0, 0)),
            pl.BlockSpec((_E, _D, _D), lambda t, b: (0, 0, 0)),
        ],
        out_specs=pl.BlockSpec((1, _TB, _D), lambda t, b: (t, b, 0)),
        out_shape=jax.ShapeDtypeStruct((2, _T, _D), jnp.float32),
        compiler_params=pltpu.CompilerParams(
            dimension_semantics=("parallel", "parallel"),
        ),
    )(xs, wgs, We)
    return (out[0], out[1])
